# baseline (speedup 1.0000x reference)
import functools

import jax
import jax.numpy as jnp
from jax import lax
from jax.experimental import pallas as pl
from jax.experimental.pallas import tpu as pltpu
from jax.experimental.pallas import tpu_sc as plsc

_N_WIDTH = 32
_N_NODES = 257
_N_SAMPLES = 4096
_N_WORKERS = 32
_SPW = _N_SAMPLES // _N_WORKERS

_F32 = jnp.float32
_I32 = jnp.int32


def _sc_body(x_hbm, w_hbm, z_hbm,
             t_hbm, dt_hbm, ddt_hbm, phi_hbm, dphi_hbm, ddphi_hbm,
             x_v, w_v,
             sp0, sp1, sd0, sd1, sdd0, sdd1,
             t_v, dt_v, ddt_v, nl_v,
             sem0, sem1):
    wid = lax.axis_index("s") * 2 + lax.axis_index("c")
    base = wid * _SPW

    pltpu.sync_copy(x_hbm.at[pl.ds(base, _SPW)], x_v)
    pltpu.sync_copy(w_hbm, w_v)
    for slab in (sp0, sp1, sd0, sd1, sdd0, sdd1):
        pltpu.sync_copy(z_hbm, slab)

    iota = lax.iota(_I32, 16)
    k_lo = iota
    k_hi = iota + 16
    fzero = jnp.zeros((16,), _F32)
    izero = jnp.zeros((16,), _I32)
    nl_v[0, :] = izero
    nl_v[1, :] = izero

    slabs = ((sp0, sd0, sdd0, sem0), (sp1, sd1, sdd1, sem1))

    def process(i, b):
        s_p, s_d, s_dd, sem = slabs[b]
        g0 = (i >> 4) << 4
        lane = i - g0
        xv = x_v[pl.ds(g0, 16)]
        lanes = lax.broadcast_in_dim(lane, (16,), ())
        dnums = lax.GatherDimensionNumbers(
            offset_dims=(), collapsed_slice_dims=(0,), start_index_map=(0,))
        xb = lax.gather(xv, lanes[:, None], dnums, slice_sizes=(1,),
                        mode=lax.GatherScatterMode.PROMISE_IN_BOUNDS)
        xs = xb * 256.0
        eli = jnp.clip((xs * 0.5).astype(_I32), 0, 127)
        nli = eli * 2
        t = xs - nli.astype(_F32) - 1.0

        p0 = 0.5 * t * (t - 1.0)
        p1 = 1.0 - t * t
        p2 = 0.5 * t * (t + 1.0)
        d0 = (t - 0.5) * 256.0
        d1 = t * -512.0
        d2 = (t + 0.5) * 256.0
        c0 = jnp.full((16,), 65536.0, _F32)
        c1 = jnp.full((16,), -131072.0, _F32)
        c2 = jnp.full((16,), 65536.0, _F32)

        nlold = nl_v[b, :]
        for j in range(3):
            pj = nlold + j
            for kk in (k_lo, k_hi):
                plsc.store_scatter(s_p, [kk, pj], fzero)
                plsc.store_scatter(s_d, [kk, pj], fzero)
                plsc.store_scatter(s_dd, [kk, pj], fzero)
        nl_v[b, :] = nli

        for j, (pv, dv, ddv) in enumerate(((p0, d0, c0), (p1, d1, c1), (p2, d2, c2))):
            pj = nli + j
            for kk in (k_lo, k_hi):
                plsc.store_scatter(s_p, [kk, pj], pv)
                plsc.store_scatter(s_d, [kk, pj], dv)
                plsc.store_scatter(s_dd, [kk, pj], ddv)

        for half, kk in enumerate((k_lo, k_hi)):
            w0 = plsc.load_gather(w_v, [kk, nli])
            w1 = plsc.load_gather(w_v, [kk, nli + 1])
            w2 = plsc.load_gather(w_v, [kk, nli + 2])
            t_v[i, pl.ds(half * 16, 16)] = w0 * p0 + w1 * p1 + w2 * p2
            dt_v[i, pl.ds(half * 16, 16)] = w0 * d0 + w1 * d1 + w2 * d2
            ddt_v[i, pl.ds(half * 16, 16)] = (w0 + w2) * 65536.0 - w1 * 131072.0

        row = (base + i) * _N_WIDTH
        pltpu.async_copy(s_p, phi_hbm.at[pl.ds(row, _N_WIDTH)], sem)
        pltpu.async_copy(s_d, dphi_hbm.at[pl.ds(row, _N_WIDTH)], sem)
        pltpu.async_copy(s_dd, ddphi_hbm.at[pl.ds(row, _N_WIDTH)], sem)

    def wait(b):
        s_p, s_d, s_dd, sem = slabs[b]
        pltpu.make_async_copy(s_p, phi_hbm.at[pl.ds(0, _N_WIDTH)], sem).wait()
        pltpu.make_async_copy(s_d, dphi_hbm.at[pl.ds(0, _N_WIDTH)], sem).wait()
        pltpu.make_async_copy(s_dd, ddphi_hbm.at[pl.ds(0, _N_WIDTH)], sem).wait()

    process(0, 0)
    process(1, 1)

    @pl.loop(2, _SPW, step=2)
    def _loop(s):
        wait(0)
        process(s, 0)
        wait(1)
        process(s + 1, 1)

    wait(0)
    wait(1)

    pltpu.sync_copy(t_v, t_hbm.at[pl.ds(base, _SPW)])
    pltpu.sync_copy(dt_v, dt_hbm.at[pl.ds(base, _SPW)])
    pltpu.sync_copy(ddt_v, ddt_hbm.at[pl.ds(base, _SPW)])


@jax.jit
def kernel(x, weight):
    mesh = plsc.VectorSubcoreMesh(core_axis_name="c", subcore_axis_name="s")
    big = jax.ShapeDtypeStruct((_N_SAMPLES * _N_WIDTH, _N_NODES), _F32)
    small = jax.ShapeDtypeStruct((_N_SAMPLES, _N_WIDTH), _F32)
    fn = pl.kernel(
        _sc_body,
        out_type=(small, small, small, big, big, big),
        mesh=mesh,
        compiler_params=pltpu.CompilerParams(
            use_tc_tiling_on_sc=False, needs_layout_passes=False),
        scratch_types=[
            pltpu.VMEM((_SPW,), _F32),
            pltpu.VMEM((_N_WIDTH, _N_NODES), _F32),
            pltpu.VMEM((_N_WIDTH, _N_NODES), _F32),
            pltpu.VMEM((_N_WIDTH, _N_NODES), _F32),
            pltpu.VMEM((_N_WIDTH, _N_NODES), _F32),
            pltpu.VMEM((_N_WIDTH, _N_NODES), _F32),
            pltpu.VMEM((_N_WIDTH, _N_NODES), _F32),
            pltpu.VMEM((_N_WIDTH, _N_NODES), _F32),
            pltpu.VMEM((_SPW, _N_WIDTH), _F32),
            pltpu.VMEM((_SPW, _N_WIDTH), _F32),
            pltpu.VMEM((_SPW, _N_WIDTH), _F32),
            pltpu.VMEM((2, 16), _I32),
            pltpu.SemaphoreType.DMA,
            pltpu.SemaphoreType.DMA,
        ],
    )
    z = jnp.zeros((_N_WIDTH, _N_NODES), _F32)
    t, dt, ddt, phi, dphi, ddphi = fn(x, weight, z)
    shp = (_N_SAMPLES, _N_WIDTH, _N_NODES)
    return (t, dt, ddt, phi.reshape(shp), dphi.reshape(shp), ddphi.reshape(shp))


# speedup vs baseline: 1.4229x; 1.4229x over previous
import functools

import jax
import jax.numpy as jnp
from jax import lax
from jax.experimental import pallas as pl
from jax.experimental.pallas import tpu as pltpu
from jax.experimental.pallas import tpu_sc as plsc

_N_WIDTH = 32
_N_NODES = 257
_N_SAMPLES = 4096
_N_WORKERS = 32
_SPW = _N_SAMPLES // _N_WORKERS

_F32 = jnp.float32
_I32 = jnp.int32


def _sc_body(x_hbm, w_hbm, z_hbm,
             t_hbm, dt_hbm, ddt_hbm, phi_hbm, dphi_hbm, ddphi_hbm,
             x_v, w_v,
             sp0, sp1, sd0, sd1, sdd0, sdd1,
             t_v, dt_v, ddt_v, nl_v,
             sem0, sem1, semt):
    wid = lax.axis_index("s") * 2 + lax.axis_index("c")
    base = wid * _SPW

    pltpu.sync_copy(x_hbm.at[pl.ds(base, _SPW)], x_v)
    pltpu.sync_copy(w_hbm, w_v)
    for slab in (sp0, sp1, sd0, sd1, sdd0, sdd1):
        pltpu.sync_copy(z_hbm, slab)

    iota = lax.iota(_I32, 16)
    k_lo = iota
    k_hi = iota + 16
    fzero = jnp.zeros((16,), _F32)
    izero = jnp.zeros((16,), _I32)
    nl_v[0, :] = izero
    nl_v[1, :] = izero

    slabs = ((sp0, sd0, sdd0, sem0), (sp1, sd1, sdd1, sem1))

    def process(i, b):
        s_p, s_d, s_dd, sem = slabs[b]
        g0 = (i >> 4) << 4
        lane = i - g0
        xv = x_v[pl.ds(g0, 16)]
        lanes = lax.broadcast_in_dim(lane, (16,), ())
        dnums = lax.GatherDimensionNumbers(
            offset_dims=(), collapsed_slice_dims=(0,), start_index_map=(0,))
        xb = lax.gather(xv, lanes[:, None], dnums, slice_sizes=(1,),
                        mode=lax.GatherScatterMode.PROMISE_IN_BOUNDS)
        xs = xb * 256.0
        eli = jnp.clip((xs * 0.5).astype(_I32), 0, 127)
        nli = eli * 2
        t = xs - nli.astype(_F32) - 1.0

        p0 = 0.5 * t * (t - 1.0)
        p1 = 1.0 - t * t
        p2 = 0.5 * t * (t + 1.0)
        d0 = (t - 0.5) * 256.0
        d1 = t * -512.0
        d2 = (t + 0.5) * 256.0
        c0 = jnp.full((16,), 65536.0, _F32)
        c1 = jnp.full((16,), -131072.0, _F32)
        c2 = jnp.full((16,), 65536.0, _F32)

        nlold = nl_v[b, :]
        for j in range(3):
            pj = nlold + j
            for kk in (k_lo, k_hi):
                plsc.store_scatter(s_p, [kk, pj], fzero)
                plsc.store_scatter(s_d, [kk, pj], fzero)
                plsc.store_scatter(s_dd, [kk, pj], fzero)
        nl_v[b, :] = nli

        for j, (pv, dv, ddv) in enumerate(((p0, d0, c0), (p1, d1, c1), (p2, d2, c2))):
            pj = nli + j
            for kk in (k_lo, k_hi):
                plsc.store_scatter(s_p, [kk, pj], pv)
                plsc.store_scatter(s_d, [kk, pj], dv)
                plsc.store_scatter(s_dd, [kk, pj], ddv)

        ci = i & 7
        for half, kk in enumerate((k_lo, k_hi)):
            w0 = plsc.load_gather(w_v, [kk, nli])
            w1 = plsc.load_gather(w_v, [kk, nli + 1])
            w2 = plsc.load_gather(w_v, [kk, nli + 2])
            t_v[ci, pl.ds(half * 16, 16)] = w0 * p0 + w1 * p1 + w2 * p2
            dt_v[ci, pl.ds(half * 16, 16)] = w0 * d0 + w1 * d1 + w2 * d2
            ddt_v[ci, pl.ds(half * 16, 16)] = (w0 + w2) * 65536.0 - w1 * 131072.0

        @pl.when(ci == 7)
        def _flush_t():
            r0 = pl.multiple_of(base + i - 7, 8)
            pltpu.sync_copy(t_v, t_hbm.at[pl.ds(r0, 8)])
            pltpu.sync_copy(dt_v, dt_hbm.at[pl.ds(r0, 8)])
            pltpu.sync_copy(ddt_v, ddt_hbm.at[pl.ds(r0, 8)])

        pltpu.async_copy(s_p, phi_hbm.at[base + i], sem)
        pltpu.async_copy(s_d, dphi_hbm.at[base + i], sem)
        pltpu.async_copy(s_dd, ddphi_hbm.at[base + i], sem)

    def wait(b):
        s_p, s_d, s_dd, sem = slabs[b]
        pltpu.make_async_copy(s_p, phi_hbm.at[0], sem).wait()
        pltpu.make_async_copy(s_d, dphi_hbm.at[0], sem).wait()
        pltpu.make_async_copy(s_dd, ddphi_hbm.at[0], sem).wait()

    process(0, 0)
    process(1, 1)

    @pl.loop(2, _SPW, step=2)
    def _loop(s):
        wait(0)
        process(s, 0)
        wait(1)
        process(s + 1, 1)

    wait(0)
    wait(1)


@jax.jit
def kernel(x, weight):
    mesh = plsc.VectorSubcoreMesh(core_axis_name="c", subcore_axis_name="s")
    big = jax.ShapeDtypeStruct((_N_SAMPLES, _N_WIDTH, _N_NODES), _F32)
    small = jax.ShapeDtypeStruct((_N_SAMPLES, _N_WIDTH), _F32)
    fn = pl.kernel(
        _sc_body,
        out_type=(small, small, small, big, big, big),
        mesh=mesh,
        compiler_params=pltpu.CompilerParams(needs_layout_passes=False),
        scratch_types=[
            pltpu.VMEM((_SPW,), _F32),
            pltpu.VMEM((_N_WIDTH, _N_NODES), _F32),
            pltpu.VMEM((_N_WIDTH, _N_NODES), _F32),
            pltpu.VMEM((_N_WIDTH, _N_NODES), _F32),
            pltpu.VMEM((_N_WIDTH, _N_NODES), _F32),
            pltpu.VMEM((_N_WIDTH, _N_NODES), _F32),
            pltpu.VMEM((_N_WIDTH, _N_NODES), _F32),
            pltpu.VMEM((_N_WIDTH, _N_NODES), _F32),
            pltpu.VMEM((8, _N_WIDTH), _F32),
            pltpu.VMEM((8, _N_WIDTH), _F32),
            pltpu.VMEM((8, _N_WIDTH), _F32),
            pltpu.VMEM((2, 16), _I32),
            pltpu.SemaphoreType.DMA,
            pltpu.SemaphoreType.DMA,
            pltpu.SemaphoreType.DMA,
        ],
    )
    z = jnp.zeros((_N_WIDTH, _N_NODES), _F32)
    return fn(x, weight, z)


# speedup vs baseline: 5.1078x; 3.5896x over previous
import functools

import jax
import jax.numpy as jnp
from jax import lax
from jax.experimental import pallas as pl
from jax.experimental.pallas import tpu as pltpu
from jax.experimental.pallas import tpu_sc as plsc

_N_WIDTH = 32
_N_NODES = 257
_N_SAMPLES = 4096
_N_WORKERS = 32
_SPW = _N_SAMPLES // _N_WORKERS
_RPW = 8
_NCHUNKS = _N_SAMPLES // 16

_F32 = jnp.float32
_I32 = jnp.int32


def _sc_body(x_hbm, w_hbm,
             t_hbm, dt_hbm, ddt_hbm, phi_hbm, dphi_hbm, ddphi_hbm,
             x_v, w_v, stag0, stag1, t_v, dt_v, ddt_v,
             sem0, sem1, semt):
    wid = lax.axis_index("s") * 2 + lax.axis_index("c")
    base = wid * _SPW

    pltpu.sync_copy(x_hbm, x_v)
    pltpu.sync_copy(w_hbm, w_v)

    iota = lax.iota(_I32, 16)
    k_lo = iota
    k_hi = iota + 16
    fzero = jnp.zeros((16,), _F32)

    dnums = lax.GatherDimensionNumbers(
        offset_dims=(), collapsed_slice_dims=(0,), start_index_map=(0,))

    def basis(xb):
        xs = xb * 256.0
        eli = jnp.clip((xs * 0.5).astype(_I32), 0, 127)
        nli = eli * 2
        t = xs - nli.astype(_F32) - 1.0
        return nli, t

    @pl.loop(0, _SPW)
    def _samples(i):
        g0 = (i >> 4) << 4
        lane = i - g0
        xv = x_v[pl.ds(base + g0, 16)]
        lanes = lax.broadcast_in_dim(lane, (16,), ())
        xb = lax.gather(xv, lanes[:, None], dnums, slice_sizes=(1,),
                        mode=lax.GatherScatterMode.PROMISE_IN_BOUNDS)
        nli, t = basis(xb)
        p0 = 0.5 * t * (t - 1.0)
        p1 = 1.0 - t * t
        p2 = 0.5 * t * (t + 1.0)
        d0 = (t - 0.5) * 256.0
        d1 = t * -512.0
        d2 = (t + 0.5) * 256.0
        il = lax.broadcast_in_dim(i, (16,), ())
        for kk in (k_lo, k_hi):
            w0 = plsc.load_gather(w_v, [kk, nli])
            w1 = plsc.load_gather(w_v, [kk, nli + 1])
            w2 = plsc.load_gather(w_v, [kk, nli + 2])
            plsc.store_scatter(t_v, [kk, il], w0 * p0 + w1 * p1 + w2 * p2)
            plsc.store_scatter(dt_v, [kk, il], w0 * d0 + w1 * d1 + w2 * d2)
            plsc.store_scatter(ddt_v, [kk, il],
                               (w0 + w2) * 65536.0 - w1 * 131072.0)

    pltpu.async_copy(t_v, t_hbm.at[:, pl.ds(base, _SPW)], semt)
    pltpu.async_copy(dt_v, dt_hbm.at[:, pl.ds(base, _SPW)], semt)
    pltpu.async_copy(ddt_v, ddt_hbm.at[:, pl.ds(base, _SPW)], semt)

    stags = ((stag0, sem0), (stag1, sem1))

    def build_and_fire(rowp, arr, out_hbm, b):
        stag, sem = stags[b]

        @pl.loop(0, _NCHUNKS)
        def _chunks(c):
            xv = x_v[pl.ds(c * 16, 16)]
            nli, t = basis(xv)
            m0 = nli == rowp
            m1 = nli == rowp - 1
            m2 = nli == rowp - 2
            if arr == 0:
                v0 = 0.5 * t * (t - 1.0)
                v1 = 1.0 - t * t
                v2 = 0.5 * t * (t + 1.0)
            elif arr == 1:
                v0 = (t - 0.5) * 256.0
                v1 = t * -512.0
                v2 = (t + 0.5) * 256.0
            else:
                v0 = jnp.full((16,), 65536.0, _F32)
                v1 = jnp.full((16,), -131072.0, _F32)
                v2 = v0
            val = (jnp.where(m0, v0, fzero) + jnp.where(m1, v1, fzero)
                   + jnp.where(m2, v2, fzero))
            off = c * 16
            for r in range(8):
                stag[r, pl.ds(off, 16)] = val

        for h in range(4):
            pltpu.async_copy(stag, out_hbm.at[rowp, pl.ds(h * 8, 8)], sem)

    def drain(b):
        stag, sem = stags[b]
        for h in range(4):
            pltpu.make_async_copy(stag, phi_hbm.at[0, pl.ds(h * 8, 8)],
                                  sem).wait()

    outs = (phi_hbm, dphi_hbm, ddphi_hbm)
    u = 0
    for r in range(_RPW):
        for arr in range(3):
            b = u & 1
            if u >= 2:
                drain(b)
            build_and_fire(wid * _RPW + r, arr, outs[arr], b)
            u += 1

    @pl.when(wid == 0)
    def _last_row():
        for arr in range(3):
            b = (u + arr) & 1
            drain(b)
            build_and_fire(_N_NODES - 1, arr, outs[arr], b)

    drain(0)
    drain(1)
    pltpu.make_async_copy(t_v, t_hbm.at[:, pl.ds(0, _SPW)], semt).wait()
    pltpu.make_async_copy(dt_v, dt_hbm.at[:, pl.ds(0, _SPW)], semt).wait()
    pltpu.make_async_copy(ddt_v, ddt_hbm.at[:, pl.ds(0, _SPW)], semt).wait()


@jax.jit
def kernel(x, weight):
    mesh = plsc.VectorSubcoreMesh(core_axis_name="c", subcore_axis_name="s")
    big = jax.ShapeDtypeStruct((_N_NODES, _N_WIDTH, _N_SAMPLES), _F32)
    small = jax.ShapeDtypeStruct((_N_WIDTH, _N_SAMPLES), _F32)
    fn = pl.kernel(
        _sc_body,
        out_type=(small, small, small, big, big, big),
        mesh=mesh,
        compiler_params=pltpu.CompilerParams(needs_layout_passes=False),
        scratch_types=[
            pltpu.VMEM((_N_SAMPLES,), _F32),
            pltpu.VMEM((_N_WIDTH, _N_NODES), _F32),
            pltpu.VMEM((8, _N_SAMPLES), _F32),
            pltpu.VMEM((8, _N_SAMPLES), _F32),
            pltpu.VMEM((_N_WIDTH, _SPW), _F32),
            pltpu.VMEM((_N_WIDTH, _SPW), _F32),
            pltpu.VMEM((_N_WIDTH, _SPW), _F32),
            pltpu.SemaphoreType.DMA,
            pltpu.SemaphoreType.DMA,
            pltpu.SemaphoreType.DMA,
        ],
    )
    t, dt, ddt, phi, dphi, ddphi = fn(x, weight)
    tr3 = lambda a: jnp.transpose(a, (2, 1, 0))
    return (t.T, dt.T, ddt.T, tr3(phi), tr3(dphi), tr3(ddphi))


# speedup vs baseline: 6.9173x; 1.3543x over previous
import functools

import jax
import jax.numpy as jnp
from jax import lax
from jax.experimental import pallas as pl
from jax.experimental.pallas import tpu as pltpu
from jax.experimental.pallas import tpu_sc as plsc

_N_WIDTH = 32
_N_NODES = 257
_N_SAMPLES = 4096
_N_WORKERS = 32
_SPW = _N_SAMPLES // _N_WORKERS
_RPW = 8
_NCHUNKS = _N_SAMPLES // 16
_PB = 8

_F32 = jnp.float32
_I32 = jnp.int32


def _sc_body(x_hbm, w_hbm,
             t_hbm, dt_hbm, ddt_hbm, ddphi_hbm,
             x_v, w_v, stag0, stag1, t_v, dt_v, ddt_v,
             sem0, sem1, semt):
    wid = lax.axis_index("s") * 2 + lax.axis_index("c")
    base = wid * _SPW

    pltpu.sync_copy(x_hbm, x_v)
    pltpu.sync_copy(w_hbm, w_v)

    iota = lax.iota(_I32, 16)
    k_lo = iota
    k_hi = iota + 16
    fzero = jnp.zeros((16,), _F32)

    dnums = lax.GatherDimensionNumbers(
        offset_dims=(), collapsed_slice_dims=(0,), start_index_map=(0,))

    def basis(xb):
        xs = xb * 256.0
        eli = jnp.clip((xs * 0.5).astype(_I32), 0, 127)
        nli = eli * 2
        t = xs - nli.astype(_F32) - 1.0
        return nli, t

    @pl.loop(0, _SPW)
    def _samples(i):
        g0 = (i >> 4) << 4
        lane = i - g0
        xv = x_v[pl.ds(base + g0, 16)]
        lanes = lax.broadcast_in_dim(lane, (16,), ())
        xb = lax.gather(xv, lanes[:, None], dnums, slice_sizes=(1,),
                        mode=lax.GatherScatterMode.PROMISE_IN_BOUNDS)
        nli, t = basis(xb)
        p0 = 0.5 * t * (t - 1.0)
        p1 = 1.0 - t * t
        p2 = 0.5 * t * (t + 1.0)
        d0 = (t - 0.5) * 256.0
        d1 = t * -512.0
        d2 = (t + 0.5) * 256.0
        il = lax.broadcast_in_dim(i, (16,), ())
        for kk in (k_lo, k_hi):
            w0 = plsc.load_gather(w_v, [kk, nli])
            w1 = plsc.load_gather(w_v, [kk, nli + 1])
            w2 = plsc.load_gather(w_v, [kk, nli + 2])
            plsc.store_scatter(t_v, [kk, il], w0 * p0 + w1 * p1 + w2 * p2)
            plsc.store_scatter(dt_v, [kk, il], w0 * d0 + w1 * d1 + w2 * d2)
            plsc.store_scatter(ddt_v, [kk, il],
                               (w0 + w2) * 65536.0 - w1 * 131072.0)

    pltpu.async_copy(t_v, t_hbm.at[:, pl.ds(base, _SPW)], semt)
    pltpu.async_copy(dt_v, dt_hbm.at[:, pl.ds(base, _SPW)], semt)
    pltpu.async_copy(ddt_v, ddt_hbm.at[:, pl.ds(base, _SPW)], semt)

    stags = ((stag0, sem0), (stag1, sem1))

    def build_and_fire(rowp, b):
        stag, sem = stags[b]

        @pl.loop(0, _NCHUNKS)
        def _chunks(c):
            xv = x_v[pl.ds(c * 16, 16)]
            nli, _ = basis(xv)
            m0 = nli == rowp
            m1 = nli == rowp - 1
            m2 = nli == rowp - 2
            v0 = jnp.full((16,), 65536.0, _F32)
            v1 = jnp.full((16,), -131072.0, _F32)
            val = (jnp.where(m0, v0, fzero) + jnp.where(m1, v1, fzero)
                   + jnp.where(m2, v0, fzero))
            off = c * 16
            for r in range(8):
                stag[r, pl.ds(off, 16)] = val

        for h in range(4):
            pltpu.async_copy(stag, ddphi_hbm.at[rowp, pl.ds(h * 8, 8)], sem)

    def drain(b):
        stag, sem = stags[b]
        for h in range(4):
            pltpu.make_async_copy(stag, ddphi_hbm.at[0, pl.ds(h * 8, 8)],
                                  sem).wait()

    for r in range(_RPW):
        b = r & 1
        if r >= 2:
            drain(b)
        build_and_fire(wid * _RPW + r, b)

    @pl.when(wid == 0)
    def _last_row():
        drain(0)
        build_and_fire(_N_NODES - 1, 0)

    drain(0)
    drain(1)
    pltpu.make_async_copy(t_v, t_hbm.at[:, pl.ds(0, _SPW)], semt).wait()
    pltpu.make_async_copy(dt_v, dt_hbm.at[:, pl.ds(0, _SPW)], semt).wait()
    pltpu.make_async_copy(ddt_v, ddt_hbm.at[:, pl.ds(0, _SPW)], semt).wait()


def _tc_body(x_ref, phi_ref, dphi_ref):
    g = pl.program_id(0)
    x = x_ref[...]
    xs = x * 256.0
    nlf = jnp.clip(jnp.floor(xs * 0.5), 0.0, 127.0) * 2.0
    t = xs - nlf - 1.0
    p0 = 0.5 * t * (t - 1.0)
    p1 = 1.0 - t * t
    p2 = 0.5 * t * (t + 1.0)
    d0 = (t - 0.5) * 256.0
    d1 = t * -512.0
    d2 = (t + 0.5) * 256.0
    nli = nlf.astype(_I32)
    prow = g * _PB + lax.broadcasted_iota(_I32, (_PB, _N_SAMPLES), 0)
    rel = prow - nli[None, :]
    m0 = rel == 0
    m1 = rel == 1
    m2 = rel == 2
    zero = jnp.zeros((), _F32)
    phi_row = jnp.where(m0, p0[None, :],
                        jnp.where(m1, p1[None, :],
                                  jnp.where(m2, p2[None, :], zero)))
    dphi_row = jnp.where(m0, d0[None, :],
                         jnp.where(m1, d1[None, :],
                                   jnp.where(m2, d2[None, :], zero)))
    shp = (_PB, _N_WIDTH, _N_SAMPLES)
    phi_ref[...] = jnp.broadcast_to(phi_row[:, None, :], shp)
    dphi_ref[...] = jnp.broadcast_to(dphi_row[:, None, :], shp)


@jax.jit
def kernel(x, weight):
    mesh = plsc.VectorSubcoreMesh(core_axis_name="c", subcore_axis_name="s")
    big = jax.ShapeDtypeStruct((_N_NODES, _N_WIDTH, _N_SAMPLES), _F32)
    small = jax.ShapeDtypeStruct((_N_WIDTH, _N_SAMPLES), _F32)
    sc_fn = pl.kernel(
        _sc_body,
        out_type=(small, small, small, big),
        mesh=mesh,
        compiler_params=pltpu.CompilerParams(needs_layout_passes=False),
        scratch_types=[
            pltpu.VMEM((_N_SAMPLES,), _F32),
            pltpu.VMEM((_N_WIDTH, _N_NODES), _F32),
            pltpu.VMEM((8, _N_SAMPLES), _F32),
            pltpu.VMEM((8, _N_SAMPLES), _F32),
            pltpu.VMEM((_N_WIDTH, _SPW), _F32),
            pltpu.VMEM((_N_WIDTH, _SPW), _F32),
            pltpu.VMEM((_N_WIDTH, _SPW), _F32),
            pltpu.SemaphoreType.DMA,
            pltpu.SemaphoreType.DMA,
            pltpu.SemaphoreType.DMA,
        ],
    )
    t, dt, ddt, ddphi = sc_fn(x, weight)

    phi, dphi = pl.pallas_call(
        _tc_body,
        grid=((_N_NODES + _PB - 1) // _PB,),
        in_specs=[pl.BlockSpec((_N_SAMPLES,), lambda g: (0,))],
        out_specs=(
            pl.BlockSpec((_PB, _N_WIDTH, _N_SAMPLES), lambda g: (g, 0, 0)),
            pl.BlockSpec((_PB, _N_WIDTH, _N_SAMPLES), lambda g: (g, 0, 0)),
        ),
        out_shape=(big, big),
    )(x)

    tr3 = lambda a: jnp.transpose(a, (2, 1, 0))
    return (t.T, dt.T, ddt.T, tr3(phi), tr3(dphi), tr3(ddphi))


# speedup vs baseline: 6.9796x; 1.0090x over previous
import functools

import jax
import jax.numpy as jnp
from jax import lax
from jax.experimental import pallas as pl
from jax.experimental.pallas import tpu as pltpu
from jax.experimental.pallas import tpu_sc as plsc

_N_WIDTH = 32
_N_NODES = 257
_N_SAMPLES = 4096
_N_WORKERS = 32
_SPW = _N_SAMPLES // _N_WORKERS
_RPW = 8
_NCHUNKS = _N_SAMPLES // 16
_PB = 8

_F32 = jnp.float32
_I32 = jnp.int32


def _sc_body(x_hbm, w_hbm,
             t_hbm, dt_hbm, ddt_hbm, ddphi_hbm,
             x_v, w_v, stag0, stag1, t_v, dt_v, ddt_v,
             sem0, sem1, semt):
    wid = lax.axis_index("s") * 2 + lax.axis_index("c")
    base = wid * _SPW

    pltpu.sync_copy(x_hbm, x_v)
    pltpu.sync_copy(w_hbm, w_v)

    iota = lax.iota(_I32, 16)
    k_lo = iota
    k_hi = iota + 16
    fzero = jnp.zeros((16,), _F32)

    dnums = lax.GatherDimensionNumbers(
        offset_dims=(), collapsed_slice_dims=(0,), start_index_map=(0,))

    def basis(xb):
        xs = xb * 256.0
        eli = jnp.clip((xs * 0.5).astype(_I32), 0, 127)
        nli = eli * 2
        t = xs - nli.astype(_F32) - 1.0
        return nli, t

    @pl.loop(0, _SPW)
    def _samples(i):
        g0 = (i >> 4) << 4
        lane = i - g0
        xv = x_v[pl.ds(base + g0, 16)]
        lanes = lax.broadcast_in_dim(lane, (16,), ())
        xb = lax.gather(xv, lanes[:, None], dnums, slice_sizes=(1,),
                        mode=lax.GatherScatterMode.PROMISE_IN_BOUNDS)
        nli, t = basis(xb)
        p0 = 0.5 * t * (t - 1.0)
        p1 = 1.0 - t * t
        p2 = 0.5 * t * (t + 1.0)
        d0 = (t - 0.5) * 256.0
        d1 = t * -512.0
        d2 = (t + 0.5) * 256.0
        il = lax.broadcast_in_dim(i, (16,), ())
        for kk in (k_lo, k_hi):
            w0 = plsc.load_gather(w_v, [kk, nli])
            w1 = plsc.load_gather(w_v, [kk, nli + 1])
            w2 = plsc.load_gather(w_v, [kk, nli + 2])
            plsc.store_scatter(t_v, [kk, il], w0 * p0 + w1 * p1 + w2 * p2)
            plsc.store_scatter(dt_v, [kk, il], w0 * d0 + w1 * d1 + w2 * d2)
            plsc.store_scatter(ddt_v, [kk, il],
                               (w0 + w2) * 65536.0 - w1 * 131072.0)

    pltpu.async_copy(t_v, t_hbm.at[:, pl.ds(base, _SPW)], semt)
    pltpu.async_copy(dt_v, dt_hbm.at[:, pl.ds(base, _SPW)], semt)
    pltpu.async_copy(ddt_v, ddt_hbm.at[:, pl.ds(base, _SPW)], semt)

    stags = ((stag0, sem0), (stag1, sem1))

    def build_and_fire(rowp, b):
        stag, sem = stags[b]

        @pl.loop(0, _NCHUNKS, unroll=4)
        def _chunks(c):
            xv = x_v[pl.ds(c * 16, 16)]
            nli, _ = basis(xv)
            m0 = nli == rowp
            m1 = nli == rowp - 1
            m2 = nli == rowp - 2
            v0 = jnp.full((16,), 65536.0, _F32)
            v1 = jnp.full((16,), -131072.0, _F32)
            val = (jnp.where(m0, v0, fzero) + jnp.where(m1, v1, fzero)
                   + jnp.where(m2, v0, fzero))
            off = c * 16
            for r in range(8):
                stag[r, pl.ds(off, 16)] = val

        for h in range(4):
            pltpu.async_copy(stag, ddphi_hbm.at[rowp, pl.ds(h * 8, 8)], sem)

    def drain(b):
        stag, sem = stags[b]
        for h in range(4):
            pltpu.make_async_copy(stag, ddphi_hbm.at[0, pl.ds(h * 8, 8)],
                                  sem).wait()

    for r in range(_RPW):
        b = r & 1
        if r >= 2:
            drain(b)
        build_and_fire(wid * _RPW + r, b)

    drain(0)

    @pl.loop(0, _SPW // 16)
    def _c256(c):
        xv = x_v[pl.ds(base + c * 16, 16)]
        nli, _ = basis(xv)
        v0 = jnp.full((16,), 65536.0, _F32)
        v1 = jnp.full((16,), -131072.0, _F32)
        val = (jnp.where(nli == _N_NODES - 1, v0, fzero)
               + jnp.where(nli == _N_NODES - 2, v1, fzero)
               + jnp.where(nli == _N_NODES - 3, v0, fzero))
        for r in range(8):
            stag0[r, pl.ds(c * 16, 16)] = val

    src256 = stag0.at[:, pl.ds(0, _SPW)]
    for h in range(4):
        pltpu.async_copy(
            src256,
            ddphi_hbm.at[_N_NODES - 1, pl.ds(h * 8, 8), pl.ds(base, _SPW)],
            sem0)

    for h in range(4):
        pltpu.make_async_copy(
            src256,
            ddphi_hbm.at[_N_NODES - 1, pl.ds(h * 8, 8), pl.ds(0, _SPW)],
            sem0).wait()
    drain(1)
    pltpu.make_async_copy(t_v, t_hbm.at[:, pl.ds(0, _SPW)], semt).wait()
    pltpu.make_async_copy(dt_v, dt_hbm.at[:, pl.ds(0, _SPW)], semt).wait()
    pltpu.make_async_copy(ddt_v, ddt_hbm.at[:, pl.ds(0, _SPW)], semt).wait()


def _tc_body(x_ref, phi_ref, dphi_ref):
    g = pl.program_id(0)
    x = x_ref[...]
    xs = x * 256.0
    nlf = jnp.clip(jnp.floor(xs * 0.5), 0.0, 127.0) * 2.0
    t = xs - nlf - 1.0
    p0 = 0.5 * t * (t - 1.0)
    p1 = 1.0 - t * t
    p2 = 0.5 * t * (t + 1.0)
    d0 = (t - 0.5) * 256.0
    d1 = t * -512.0
    d2 = (t + 0.5) * 256.0
    nli = nlf.astype(_I32)
    prow = g * _PB + lax.broadcasted_iota(_I32, (_PB, _N_SAMPLES), 0)
    rel = prow - nli[None, :]
    m0 = rel == 0
    m1 = rel == 1
    m2 = rel == 2
    zero = jnp.zeros((), _F32)
    phi_row = jnp.where(m0, p0[None, :],
                        jnp.where(m1, p1[None, :],
                                  jnp.where(m2, p2[None, :], zero)))
    dphi_row = jnp.where(m0, d0[None, :],
                         jnp.where(m1, d1[None, :],
                                   jnp.where(m2, d2[None, :], zero)))
    shp = (_PB, _N_WIDTH, _N_SAMPLES)
    phi_ref[...] = jnp.broadcast_to(phi_row[:, None, :], shp)
    dphi_ref[...] = jnp.broadcast_to(dphi_row[:, None, :], shp)


@jax.jit
def kernel(x, weight):
    mesh = plsc.VectorSubcoreMesh(core_axis_name="c", subcore_axis_name="s")
    big = jax.ShapeDtypeStruct((_N_NODES, _N_WIDTH, _N_SAMPLES), _F32)
    small = jax.ShapeDtypeStruct((_N_WIDTH, _N_SAMPLES), _F32)
    sc_fn = pl.kernel(
        _sc_body,
        out_type=(small, small, small, big),
        mesh=mesh,
        compiler_params=pltpu.CompilerParams(needs_layout_passes=False),
        scratch_types=[
            pltpu.VMEM((_N_SAMPLES,), _F32),
            pltpu.VMEM((_N_WIDTH, _N_NODES), _F32),
            pltpu.VMEM((8, _N_SAMPLES), _F32),
            pltpu.VMEM((8, _N_SAMPLES), _F32),
            pltpu.VMEM((_N_WIDTH, _SPW), _F32),
            pltpu.VMEM((_N_WIDTH, _SPW), _F32),
            pltpu.VMEM((_N_WIDTH, _SPW), _F32),
            pltpu.SemaphoreType.DMA,
            pltpu.SemaphoreType.DMA,
            pltpu.SemaphoreType.DMA,
        ],
    )
    t, dt, ddt, ddphi = sc_fn(x, weight)

    phi, dphi = pl.pallas_call(
        _tc_body,
        grid=((_N_NODES + _PB - 1) // _PB,),
        in_specs=[pl.BlockSpec((_N_SAMPLES,), lambda g: (0,))],
        out_specs=(
            pl.BlockSpec((_PB, _N_WIDTH, _N_SAMPLES), lambda g: (g, 0, 0)),
            pl.BlockSpec((_PB, _N_WIDTH, _N_SAMPLES), lambda g: (g, 0, 0)),
        ),
        out_shape=(big, big),
    )(x)

    tr3 = lambda a: jnp.transpose(a, (2, 1, 0))
    return (t.T, dt.T, ddt.T, tr3(phi), tr3(dphi), tr3(ddphi))


# speedup vs baseline: 7.0633x; 1.0120x over previous
import functools

import jax
import jax.numpy as jnp
from jax import lax
from jax.experimental import pallas as pl
from jax.experimental.pallas import tpu as pltpu
from jax.experimental.pallas import tpu_sc as plsc

_N_WIDTH = 32
_N_NODES = 257
_N_SAMPLES = 4096
_N_WORKERS = 32
_SPW = _N_SAMPLES // _N_WORKERS
_RPW = 8
_NCHUNKS = _N_SAMPLES // 16
_PB = 8

_F32 = jnp.float32
_I32 = jnp.int32


def _sc_body(x_hbm, w_hbm,
             t_hbm, dt_hbm, ddt_hbm, ddphi_hbm,
             x_v, w_v, stag0, stag1, stag2, t_v, dt_v, ddt_v,
             sem0, sem1, sem2, semt):
    wid = lax.axis_index("s") * 2 + lax.axis_index("c")
    base = wid * _SPW

    pltpu.sync_copy(x_hbm, x_v)
    pltpu.sync_copy(w_hbm, w_v)

    iota = lax.iota(_I32, 16)
    k_lo = iota
    k_hi = iota + 16
    fzero = jnp.zeros((16,), _F32)

    dnums = lax.GatherDimensionNumbers(
        offset_dims=(), collapsed_slice_dims=(0,), start_index_map=(0,))

    def basis(xb):
        xs = xb * 256.0
        eli = jnp.clip((xs * 0.5).astype(_I32), 0, 127)
        nli = eli * 2
        t = xs - nli.astype(_F32) - 1.0
        return nli, t

    @pl.loop(0, _SPW)
    def _samples(i):
        g0 = (i >> 4) << 4
        lane = i - g0
        xv = x_v[pl.ds(base + g0, 16)]
        lanes = lax.broadcast_in_dim(lane, (16,), ())
        xb = lax.gather(xv, lanes[:, None], dnums, slice_sizes=(1,),
                        mode=lax.GatherScatterMode.PROMISE_IN_BOUNDS)
        nli, t = basis(xb)
        p0 = 0.5 * t * (t - 1.0)
        p1 = 1.0 - t * t
        p2 = 0.5 * t * (t + 1.0)
        d0 = (t - 0.5) * 256.0
        d1 = t * -512.0
        d2 = (t + 0.5) * 256.0
        il = lax.broadcast_in_dim(i, (16,), ())
        for kk in (k_lo, k_hi):
            w0 = plsc.load_gather(w_v, [kk, nli])
            w1 = plsc.load_gather(w_v, [kk, nli + 1])
            w2 = plsc.load_gather(w_v, [kk, nli + 2])
            plsc.store_scatter(t_v, [kk, il], w0 * p0 + w1 * p1 + w2 * p2)
            plsc.store_scatter(dt_v, [kk, il], w0 * d0 + w1 * d1 + w2 * d2)
            plsc.store_scatter(ddt_v, [kk, il],
                               (w0 + w2) * 65536.0 - w1 * 131072.0)

    pltpu.async_copy(t_v, t_hbm.at[:, pl.ds(base, _SPW)], semt)
    pltpu.async_copy(dt_v, dt_hbm.at[:, pl.ds(base, _SPW)], semt)
    pltpu.async_copy(ddt_v, ddt_hbm.at[:, pl.ds(base, _SPW)], semt)

    stags = ((stag0, sem0), (stag1, sem1), (stag2, sem2))

    def build_and_fire(rowp, b):
        stag, sem = stags[b]

        @pl.loop(0, _NCHUNKS, unroll=4)
        def _chunks(c):
            xv = x_v[pl.ds(c * 16, 16)]
            nli, _ = basis(xv)
            m0 = nli == rowp
            m1 = nli == rowp - 1
            m2 = nli == rowp - 2
            v0 = jnp.full((16,), 65536.0, _F32)
            v1 = jnp.full((16,), -131072.0, _F32)
            val = (jnp.where(m0, v0, fzero) + jnp.where(m1, v1, fzero)
                   + jnp.where(m2, v0, fzero))
            off = c * 16
            for r in range(8):
                stag[r, pl.ds(off, 16)] = val

        for h in range(4):
            pltpu.async_copy(stag, ddphi_hbm.at[rowp, pl.ds(h * 8, 8)], sem)

    def drain(b):
        stag, sem = stags[b]
        for h in range(4):
            pltpu.make_async_copy(stag, ddphi_hbm.at[0, pl.ds(h * 8, 8)],
                                  sem).wait()

    for r in range(_RPW):
        b = r % 3
        if r >= 3:
            drain(b)
        build_and_fire(wid * _RPW + r, b)

    drain(0)

    @pl.loop(0, _SPW // 16)
    def _c256(c):
        xv = x_v[pl.ds(base + c * 16, 16)]
        nli, _ = basis(xv)
        v0 = jnp.full((16,), 65536.0, _F32)
        v1 = jnp.full((16,), -131072.0, _F32)
        val = (jnp.where(nli == _N_NODES - 1, v0, fzero)
               + jnp.where(nli == _N_NODES - 2, v1, fzero)
               + jnp.where(nli == _N_NODES - 3, v0, fzero))
        for r in range(8):
            stag0[r, pl.ds(c * 16, 16)] = val

    src256 = stag0.at[:, pl.ds(0, _SPW)]
    for h in range(4):
        pltpu.async_copy(
            src256,
            ddphi_hbm.at[_N_NODES - 1, pl.ds(h * 8, 8), pl.ds(base, _SPW)],
            sem0)

    for h in range(4):
        pltpu.make_async_copy(
            src256,
            ddphi_hbm.at[_N_NODES - 1, pl.ds(h * 8, 8), pl.ds(0, _SPW)],
            sem0).wait()
    drain(1)
    drain(2)
    pltpu.make_async_copy(t_v, t_hbm.at[:, pl.ds(0, _SPW)], semt).wait()
    pltpu.make_async_copy(dt_v, dt_hbm.at[:, pl.ds(0, _SPW)], semt).wait()
    pltpu.make_async_copy(ddt_v, ddt_hbm.at[:, pl.ds(0, _SPW)], semt).wait()


def _tc_body(x_ref, phi_ref, dphi_ref):
    g = pl.program_id(0)
    x = x_ref[...]
    xs = x * 256.0
    nlf = jnp.clip(jnp.floor(xs * 0.5), 0.0, 127.0) * 2.0
    t = xs - nlf - 1.0
    p0 = 0.5 * t * (t - 1.0)
    p1 = 1.0 - t * t
    p2 = 0.5 * t * (t + 1.0)
    d0 = (t - 0.5) * 256.0
    d1 = t * -512.0
    d2 = (t + 0.5) * 256.0
    nli = nlf.astype(_I32)
    prow = g * _PB + lax.broadcasted_iota(_I32, (_PB, _N_SAMPLES), 0)
    rel = prow - nli[None, :]
    m0 = rel == 0
    m1 = rel == 1
    m2 = rel == 2
    zero = jnp.zeros((), _F32)
    phi_row = jnp.where(m0, p0[None, :],
                        jnp.where(m1, p1[None, :],
                                  jnp.where(m2, p2[None, :], zero)))
    dphi_row = jnp.where(m0, d0[None, :],
                         jnp.where(m1, d1[None, :],
                                   jnp.where(m2, d2[None, :], zero)))
    shp = (_PB, _N_WIDTH, _N_SAMPLES)
    phi_ref[...] = jnp.broadcast_to(phi_row[:, None, :], shp)
    dphi_ref[...] = jnp.broadcast_to(dphi_row[:, None, :], shp)


@jax.jit
def kernel(x, weight):
    mesh = plsc.VectorSubcoreMesh(core_axis_name="c", subcore_axis_name="s")
    big = jax.ShapeDtypeStruct((_N_NODES, _N_WIDTH, _N_SAMPLES), _F32)
    small = jax.ShapeDtypeStruct((_N_WIDTH, _N_SAMPLES), _F32)
    sc_fn = pl.kernel(
        _sc_body,
        out_type=(small, small, small, big),
        mesh=mesh,
        compiler_params=pltpu.CompilerParams(needs_layout_passes=False),
        scratch_types=[
            pltpu.VMEM((_N_SAMPLES,), _F32),
            pltpu.VMEM((_N_WIDTH, _N_NODES), _F32),
            pltpu.VMEM((8, _N_SAMPLES), _F32),
            pltpu.VMEM((8, _N_SAMPLES), _F32),
            pltpu.VMEM((8, _N_SAMPLES), _F32),
            pltpu.VMEM((_N_WIDTH, _SPW), _F32),
            pltpu.VMEM((_N_WIDTH, _SPW), _F32),
            pltpu.VMEM((_N_WIDTH, _SPW), _F32),
            pltpu.SemaphoreType.DMA,
            pltpu.SemaphoreType.DMA,
            pltpu.SemaphoreType.DMA,
            pltpu.SemaphoreType.DMA,
        ],
    )
    t, dt, ddt, ddphi = sc_fn(x, weight)

    phi, dphi = pl.pallas_call(
        _tc_body,
        grid=((_N_NODES + _PB - 1) // _PB,),
        in_specs=[pl.BlockSpec((_N_SAMPLES,), lambda g: (0,))],
        out_specs=(
            pl.BlockSpec((_PB, _N_WIDTH, _N_SAMPLES), lambda g: (g, 0, 0)),
            pl.BlockSpec((_PB, _N_WIDTH, _N_SAMPLES), lambda g: (g, 0, 0)),
        ),
        out_shape=(big, big),
    )(x)

    tr3 = lambda a: jnp.transpose(a, (2, 1, 0))
    return (t.T, dt.T, ddt.T, tr3(phi), tr3(dphi), tr3(ddphi))


# speedup vs baseline: 7.1552x; 1.0130x over previous
import functools

import jax
import jax.numpy as jnp
from jax import lax
from jax.experimental import pallas as pl
from jax.experimental.pallas import tpu as pltpu
from jax.experimental.pallas import tpu_sc as plsc

_N_WIDTH = 32
_N_NODES = 257
_N_SAMPLES = 4096
_N_WORKERS = 32
_SPW = _N_SAMPLES // _N_WORKERS
_RPW = 8
_NCHUNKS = _N_SAMPLES // 16
_PB = 8

_F32 = jnp.float32
_I32 = jnp.int32


def _sc_body(x_hbm, w_hbm,
             t_hbm, dt_hbm, ddt_hbm, ddphi_hbm,
             x_v, w_v, stag0, stag1, stag2, t_v, dt_v, ddt_v,
             sem0, sem1, sem2, semt):
    wid = lax.axis_index("s") * 2 + lax.axis_index("c")
    base = wid * _SPW

    pltpu.sync_copy(x_hbm, x_v)

    iota = lax.iota(_I32, 16)
    k_lo = iota
    k_hi = iota + 16
    fzero = jnp.zeros((16,), _F32)

    dnums = lax.GatherDimensionNumbers(
        offset_dims=(), collapsed_slice_dims=(0,), start_index_map=(0,))

    def basis(xb):
        xs = xb * 256.0
        eli = jnp.clip((xs * 0.5).astype(_I32), 0, 127)
        nli = eli * 2
        t = xs - nli.astype(_F32) - 1.0
        return nli, t

    stags = ((stag0, sem0), (stag1, sem1), (stag2, sem2))

    def build_and_fire(rowp, b):
        stag, sem = stags[b]

        @pl.loop(0, _NCHUNKS, unroll=4)
        def _chunks(c):
            xv = x_v[pl.ds(c * 16, 16)]
            nli, _ = basis(xv)
            m0 = nli == rowp
            m1 = nli == rowp - 1
            m2 = nli == rowp - 2
            v0 = jnp.full((16,), 65536.0, _F32)
            v1 = jnp.full((16,), -131072.0, _F32)
            val = (jnp.where(m0, v0, fzero) + jnp.where(m1, v1, fzero)
                   + jnp.where(m2, v0, fzero))
            off = c * 16
            for r in range(8):
                stag[r, pl.ds(off, 16)] = val

        for h in range(4):
            pltpu.async_copy(stag, ddphi_hbm.at[rowp, pl.ds(h * 8, 8)], sem)

    def drain(b):
        stag, sem = stags[b]
        for h in range(4):
            pltpu.make_async_copy(stag, ddphi_hbm.at[0, pl.ds(h * 8, 8)],
                                  sem).wait()

    for r in range(_RPW):
        b = r % 3
        if r >= 3:
            drain(b)
        build_and_fire(wid * _RPW + r, b)

    drain(0)

    @pl.loop(0, _SPW // 16)
    def _c256(c):
        xv = x_v[pl.ds(base + c * 16, 16)]
        nli, _ = basis(xv)
        v0 = jnp.full((16,), 65536.0, _F32)
        v1 = jnp.full((16,), -131072.0, _F32)
        val = (jnp.where(nli == _N_NODES - 1, v0, fzero)
               + jnp.where(nli == _N_NODES - 2, v1, fzero)
               + jnp.where(nli == _N_NODES - 3, v0, fzero))
        for r in range(8):
            stag0[r, pl.ds(c * 16, 16)] = val

    src256 = stag0.at[:, pl.ds(0, _SPW)]
    for h in range(4):
        pltpu.async_copy(
            src256,
            ddphi_hbm.at[_N_NODES - 1, pl.ds(h * 8, 8), pl.ds(base, _SPW)],
            sem0)

    pltpu.sync_copy(w_hbm, w_v)

    @pl.loop(0, _SPW)
    def _samples(i):
        g0 = (i >> 4) << 4
        lane = i - g0
        xv = x_v[pl.ds(base + g0, 16)]
        lanes = lax.broadcast_in_dim(lane, (16,), ())
        xb = lax.gather(xv, lanes[:, None], dnums, slice_sizes=(1,),
                        mode=lax.GatherScatterMode.PROMISE_IN_BOUNDS)
        nli, t = basis(xb)
        p0 = 0.5 * t * (t - 1.0)
        p1 = 1.0 - t * t
        p2 = 0.5 * t * (t + 1.0)
        d0 = (t - 0.5) * 256.0
        d1 = t * -512.0
        d2 = (t + 0.5) * 256.0
        il = lax.broadcast_in_dim(i, (16,), ())
        for kk in (k_lo, k_hi):
            w0 = plsc.load_gather(w_v, [kk, nli])
            w1 = plsc.load_gather(w_v, [kk, nli + 1])
            w2 = plsc.load_gather(w_v, [kk, nli + 2])
            plsc.store_scatter(t_v, [kk, il], w0 * p0 + w1 * p1 + w2 * p2)
            plsc.store_scatter(dt_v, [kk, il], w0 * d0 + w1 * d1 + w2 * d2)
            plsc.store_scatter(ddt_v, [kk, il],
                               (w0 + w2) * 65536.0 - w1 * 131072.0)

    pltpu.async_copy(t_v, t_hbm.at[:, pl.ds(base, _SPW)], semt)
    pltpu.async_copy(dt_v, dt_hbm.at[:, pl.ds(base, _SPW)], semt)
    pltpu.async_copy(ddt_v, ddt_hbm.at[:, pl.ds(base, _SPW)], semt)

    for h in range(4):
        pltpu.make_async_copy(
            src256,
            ddphi_hbm.at[_N_NODES - 1, pl.ds(h * 8, 8), pl.ds(0, _SPW)],
            sem0).wait()
    drain(1)
    drain(2)
    pltpu.make_async_copy(t_v, t_hbm.at[:, pl.ds(0, _SPW)], semt).wait()
    pltpu.make_async_copy(dt_v, dt_hbm.at[:, pl.ds(0, _SPW)], semt).wait()
    pltpu.make_async_copy(ddt_v, ddt_hbm.at[:, pl.ds(0, _SPW)], semt).wait()


def _tc_body(x_ref, phi_ref, dphi_ref):
    g = pl.program_id(0)
    x = x_ref[...]
    xs = x * 256.0
    nlf = jnp.clip(jnp.floor(xs * 0.5), 0.0, 127.0) * 2.0
    t = xs - nlf - 1.0
    p0 = 0.5 * t * (t - 1.0)
    p1 = 1.0 - t * t
    p2 = 0.5 * t * (t + 1.0)
    d0 = (t - 0.5) * 256.0
    d1 = t * -512.0
    d2 = (t + 0.5) * 256.0
    nli = nlf.astype(_I32)
    prow = g * _PB + lax.broadcasted_iota(_I32, (_PB, _N_SAMPLES), 0)
    rel = prow - nli[None, :]
    m0 = rel == 0
    m1 = rel == 1
    m2 = rel == 2
    zero = jnp.zeros((), _F32)
    phi_row = jnp.where(m0, p0[None, :],
                        jnp.where(m1, p1[None, :],
                                  jnp.where(m2, p2[None, :], zero)))
    dphi_row = jnp.where(m0, d0[None, :],
                         jnp.where(m1, d1[None, :],
                                   jnp.where(m2, d2[None, :], zero)))
    shp = (_PB, _N_WIDTH, _N_SAMPLES)
    phi_ref[...] = jnp.broadcast_to(phi_row[:, None, :], shp)
    dphi_ref[...] = jnp.broadcast_to(dphi_row[:, None, :], shp)


@jax.jit
def kernel(x, weight):
    mesh = plsc.VectorSubcoreMesh(core_axis_name="c", subcore_axis_name="s")
    big = jax.ShapeDtypeStruct((_N_NODES, _N_WIDTH, _N_SAMPLES), _F32)
    small = jax.ShapeDtypeStruct((_N_WIDTH, _N_SAMPLES), _F32)
    sc_fn = pl.kernel(
        _sc_body,
        out_type=(small, small, small, big),
        mesh=mesh,
        compiler_params=pltpu.CompilerParams(needs_layout_passes=False),
        scratch_types=[
            pltpu.VMEM((_N_SAMPLES,), _F32),
            pltpu.VMEM((_N_WIDTH, _N_NODES), _F32),
            pltpu.VMEM((8, _N_SAMPLES), _F32),
            pltpu.VMEM((8, _N_SAMPLES), _F32),
            pltpu.VMEM((8, _N_SAMPLES), _F32),
            pltpu.VMEM((_N_WIDTH, _SPW), _F32),
            pltpu.VMEM((_N_WIDTH, _SPW), _F32),
            pltpu.VMEM((_N_WIDTH, _SPW), _F32),
            pltpu.SemaphoreType.DMA,
            pltpu.SemaphoreType.DMA,
            pltpu.SemaphoreType.DMA,
            pltpu.SemaphoreType.DMA,
        ],
    )
    t, dt, ddt, ddphi = sc_fn(x, weight)

    phi, dphi = pl.pallas_call(
        _tc_body,
        grid=((_N_NODES + _PB - 1) // _PB,),
        in_specs=[pl.BlockSpec((_N_SAMPLES,), lambda g: (0,))],
        out_specs=(
            pl.BlockSpec((_PB, _N_WIDTH, _N_SAMPLES), lambda g: (g, 0, 0)),
            pl.BlockSpec((_PB, _N_WIDTH, _N_SAMPLES), lambda g: (g, 0, 0)),
        ),
        out_shape=(big, big),
    )(x)

    tr3 = lambda a: jnp.transpose(a, (2, 1, 0))
    return (t.T, dt.T, ddt.T, tr3(phi), tr3(dphi), tr3(ddphi))


# speedup vs baseline: 7.1665x; 1.0016x over previous
import functools

import jax
import jax.numpy as jnp
from jax import lax
from jax.experimental import pallas as pl
from jax.experimental.pallas import tpu as pltpu
from jax.experimental.pallas import tpu_sc as plsc

_N_WIDTH = 32
_N_NODES = 257
_N_SAMPLES = 4096
_N_WORKERS = 32
_SPW = _N_SAMPLES // _N_WORKERS
_RPW = 8
_NCHUNKS = _N_SAMPLES // 16
_PB = 8

_F32 = jnp.float32
_I32 = jnp.int32


def _sc_body(x_hbm, ddphi_hbm, x_v, stag0, stag1, stag2,
             sem0, sem1, sem2):
    wid = lax.axis_index("s") * 2 + lax.axis_index("c")
    base = wid * _SPW

    pltpu.sync_copy(x_hbm, x_v)

    fzero = jnp.zeros((16,), _F32)

    def nodes(xb):
        xs = xb * 256.0
        eli = jnp.clip((xs * 0.5).astype(_I32), 0, 127)
        return eli * 2

    stags = ((stag0, sem0), (stag1, sem1), (stag2, sem2))

    def build_and_fire(rowp, b):
        stag, sem = stags[b]

        @pl.loop(0, _NCHUNKS, unroll=4)
        def _chunks(c):
            nli = nodes(x_v[pl.ds(c * 16, 16)])
            m0 = nli == rowp
            m1 = nli == rowp - 1
            m2 = nli == rowp - 2
            v0 = jnp.full((16,), 65536.0, _F32)
            v1 = jnp.full((16,), -131072.0, _F32)
            val = (jnp.where(m0, v0, fzero) + jnp.where(m1, v1, fzero)
                   + jnp.where(m2, v0, fzero))
            off = c * 16
            for r in range(8):
                stag[r, pl.ds(off, 16)] = val

        for h in range(4):
            pltpu.async_copy(stag, ddphi_hbm.at[rowp, pl.ds(h * 8, 8)], sem)

    def drain(b):
        stag, sem = stags[b]
        for h in range(4):
            pltpu.make_async_copy(stag, ddphi_hbm.at[0, pl.ds(h * 8, 8)],
                                  sem).wait()

    for r in range(_RPW):
        b = r % 3
        if r >= 3:
            drain(b)
        build_and_fire(wid * _RPW + r, b)

    drain(0)

    @pl.loop(0, _SPW // 16)
    def _c256(c):
        nli = nodes(x_v[pl.ds(base + c * 16, 16)])
        v0 = jnp.full((16,), 65536.0, _F32)
        v1 = jnp.full((16,), -131072.0, _F32)
        val = (jnp.where(nli == _N_NODES - 1, v0, fzero)
               + jnp.where(nli == _N_NODES - 2, v1, fzero)
               + jnp.where(nli == _N_NODES - 3, v0, fzero))
        for r in range(8):
            stag0[r, pl.ds(c * 16, 16)] = val

    src256 = stag0.at[:, pl.ds(0, _SPW)]
    for h in range(4):
        pltpu.async_copy(
            src256,
            ddphi_hbm.at[_N_NODES - 1, pl.ds(h * 8, 8), pl.ds(base, _SPW)],
            sem0)

    for h in range(4):
        pltpu.make_async_copy(
            src256,
            ddphi_hbm.at[_N_NODES - 1, pl.ds(h * 8, 8), pl.ds(0, _SPW)],
            sem0).wait()
    drain(1)
    drain(2)


def _tc_body(x_ref, w_ref, phi_ref, dphi_ref, t_ref, dt_ref, ddt_ref):
    g = pl.program_id(0)
    x = x_ref[...]
    xs = x * 256.0
    nlf = jnp.clip(jnp.floor(xs * 0.5), 0.0, 127.0) * 2.0
    t = xs - nlf - 1.0
    p0 = 0.5 * t * (t - 1.0)
    p1 = 1.0 - t * t
    p2 = 0.5 * t * (t + 1.0)
    d0 = (t - 0.5) * 256.0
    d1 = t * -512.0
    d2 = (t + 0.5) * 256.0
    nli = nlf.astype(_I32)
    prow = g * _PB + lax.broadcasted_iota(_I32, (_PB, _N_SAMPLES), 0)
    rel = prow - nli[None, :]
    m0 = rel == 0
    m1 = rel == 1
    m2 = rel == 2
    zero = jnp.zeros((), _F32)
    phi_row = jnp.where(m0, p0[None, :],
                        jnp.where(m1, p1[None, :],
                                  jnp.where(m2, p2[None, :], zero)))
    dphi_row = jnp.where(m0, d0[None, :],
                         jnp.where(m1, d1[None, :],
                                   jnp.where(m2, d2[None, :], zero)))
    ddphi_row = (jnp.where(m0, 65536.0, zero) + jnp.where(m1, -131072.0, zero)
                 + jnp.where(m2, 65536.0, zero))
    shp = (_PB, _N_WIDTH, _N_SAMPLES)
    phi_ref[...] = jnp.broadcast_to(phi_row[:, None, :], shp)
    dphi_ref[...] = jnp.broadcast_to(dphi_row[:, None, :], shp)

    wb = w_ref[...]
    col = g * _PB + lax.broadcasted_iota(_I32, (_PB, _N_WIDTH), 0)
    wb = jnp.where(col < _N_NODES, wb, zero)
    dn = (((0,), (0,)), ((), ()))
    pt = lax.dot_general(wb, phi_row, dn, preferred_element_type=_F32)
    pdt = lax.dot_general(wb, dphi_row, dn, preferred_element_type=_F32)
    pddt = lax.dot_general(wb, ddphi_row, dn, preferred_element_type=_F32)

    @pl.when(g == 0)
    def _init():
        t_ref[...] = pt
        dt_ref[...] = pdt
        ddt_ref[...] = pddt

    @pl.when(g > 0)
    def _acc():
        t_ref[...] = t_ref[...] + pt
        dt_ref[...] = dt_ref[...] + pdt
        ddt_ref[...] = ddt_ref[...] + pddt


@jax.jit
def kernel(x, weight):
    mesh = plsc.VectorSubcoreMesh(core_axis_name="c", subcore_axis_name="s")
    big = jax.ShapeDtypeStruct((_N_NODES, _N_WIDTH, _N_SAMPLES), _F32)
    small = jax.ShapeDtypeStruct((_N_WIDTH, _N_SAMPLES), _F32)
    sc_fn = pl.kernel(
        _sc_body,
        out_type=big,
        mesh=mesh,
        compiler_params=pltpu.CompilerParams(needs_layout_passes=False),
        scratch_types=[
            pltpu.VMEM((_N_SAMPLES,), _F32),
            pltpu.VMEM((8, _N_SAMPLES), _F32),
            pltpu.VMEM((8, _N_SAMPLES), _F32),
            pltpu.VMEM((8, _N_SAMPLES), _F32),
            pltpu.SemaphoreType.DMA,
            pltpu.SemaphoreType.DMA,
            pltpu.SemaphoreType.DMA,
        ],
    )
    ddphi = sc_fn(x)

    nsteps = (_N_NODES + _PB - 1) // _PB
    phi, dphi, t, dt, ddt = pl.pallas_call(
        _tc_body,
        grid=(nsteps,),
        in_specs=[
            pl.BlockSpec((_N_SAMPLES,), lambda g: (0,)),
            pl.BlockSpec((_PB, _N_WIDTH), lambda g: (g, 0)),
        ],
        out_specs=(
            pl.BlockSpec((_PB, _N_WIDTH, _N_SAMPLES), lambda g: (g, 0, 0)),
            pl.BlockSpec((_PB, _N_WIDTH, _N_SAMPLES), lambda g: (g, 0, 0)),
            pl.BlockSpec((_N_WIDTH, _N_SAMPLES), lambda g: (0, 0)),
            pl.BlockSpec((_N_WIDTH, _N_SAMPLES), lambda g: (0, 0)),
            pl.BlockSpec((_N_WIDTH, _N_SAMPLES), lambda g: (0, 0)),
        ),
        out_shape=(big, big, small, small, small),
    )(x, weight.T)

    tr3 = lambda a: jnp.transpose(a, (2, 1, 0))
    return (t.T, dt.T, ddt.T, tr3(phi), tr3(dphi), tr3(ddphi))


# speedup vs baseline: 7.2390x; 1.0101x over previous
import functools

import jax
import jax.numpy as jnp
from jax import lax
from jax.experimental import pallas as pl
from jax.experimental.pallas import tpu as pltpu
from jax.experimental.pallas import tpu_sc as plsc

_N_WIDTH = 32
_N_NODES = 257
_N_SAMPLES = 4096
_N_WORKERS = 32
_SPW = _N_SAMPLES // _N_WORKERS
_RPW = 8
_NCHUNKS = _N_SAMPLES // 16
_PB = 16

_F32 = jnp.float32
_I32 = jnp.int32


def _sc_body(x_hbm, ddphi_hbm, x_v, stag0, stag1, stag2,
             sem0, sem1, sem2):
    wid = lax.axis_index("s") * 2 + lax.axis_index("c")
    base = wid * _SPW

    pltpu.sync_copy(x_hbm, x_v)

    fzero = jnp.zeros((16,), _F32)

    def nodes(xb):
        xs = xb * 256.0
        eli = jnp.clip((xs * 0.5).astype(_I32), 0, 127)
        return eli * 2

    stags = ((stag0, sem0), (stag1, sem1), (stag2, sem2))

    def build_and_fire(rowp, b):
        stag, sem = stags[b]

        @pl.loop(0, _NCHUNKS, unroll=4)
        def _chunks(c):
            nli = nodes(x_v[pl.ds(c * 16, 16)])
            m0 = nli == rowp
            m1 = nli == rowp - 1
            m2 = nli == rowp - 2
            v0 = jnp.full((16,), 65536.0, _F32)
            v1 = jnp.full((16,), -131072.0, _F32)
            val = (jnp.where(m0, v0, fzero) + jnp.where(m1, v1, fzero)
                   + jnp.where(m2, v0, fzero))
            off = c * 16
            for r in range(8):
                stag[r, pl.ds(off, 16)] = val

        for h in range(4):
            pltpu.async_copy(stag, ddphi_hbm.at[rowp, pl.ds(h * 8, 8)], sem)

    def drain(b):
        stag, sem = stags[b]
        for h in range(4):
            pltpu.make_async_copy(stag, ddphi_hbm.at[0, pl.ds(h * 8, 8)],
                                  sem).wait()

    for r in range(_RPW):
        b = r % 3
        if r >= 3:
            drain(b)
        build_and_fire(wid * _RPW + r, b)

    drain(0)

    @pl.loop(0, _SPW // 16)
    def _c256(c):
        nli = nodes(x_v[pl.ds(base + c * 16, 16)])
        v0 = jnp.full((16,), 65536.0, _F32)
        v1 = jnp.full((16,), -131072.0, _F32)
        val = (jnp.where(nli == _N_NODES - 1, v0, fzero)
               + jnp.where(nli == _N_NODES - 2, v1, fzero)
               + jnp.where(nli == _N_NODES - 3, v0, fzero))
        for r in range(8):
            stag0[r, pl.ds(c * 16, 16)] = val

    src256 = stag0.at[:, pl.ds(0, _SPW)]
    for h in range(4):
        pltpu.async_copy(
            src256,
            ddphi_hbm.at[_N_NODES - 1, pl.ds(h * 8, 8), pl.ds(base, _SPW)],
            sem0)

    for h in range(4):
        pltpu.make_async_copy(
            src256,
            ddphi_hbm.at[_N_NODES - 1, pl.ds(h * 8, 8), pl.ds(0, _SPW)],
            sem0).wait()
    drain(1)
    drain(2)


def _tc_body(x_ref, w_ref, phi_ref, dphi_ref, t_ref, dt_ref, ddt_ref):
    g = pl.program_id(0)
    x = x_ref[...]
    xs = x * 256.0
    nlf = jnp.clip(jnp.floor(xs * 0.5), 0.0, 127.0) * 2.0
    t = xs - nlf - 1.0
    p0 = 0.5 * t * (t - 1.0)
    p1 = 1.0 - t * t
    p2 = 0.5 * t * (t + 1.0)
    d0 = (t - 0.5) * 256.0
    d1 = t * -512.0
    d2 = (t + 0.5) * 256.0
    nli = nlf.astype(_I32)
    prow = g * _PB + lax.broadcasted_iota(_I32, (_PB, _N_SAMPLES), 0)
    rel = prow - nli[None, :]
    m0 = rel == 0
    m1 = rel == 1
    m2 = rel == 2
    zero = jnp.zeros((), _F32)
    phi_row = jnp.where(m0, p0[None, :],
                        jnp.where(m1, p1[None, :],
                                  jnp.where(m2, p2[None, :], zero)))
    dphi_row = jnp.where(m0, d0[None, :],
                         jnp.where(m1, d1[None, :],
                                   jnp.where(m2, d2[None, :], zero)))
    ddphi_row = (jnp.where(m0, 65536.0, zero) + jnp.where(m1, -131072.0, zero)
                 + jnp.where(m2, 65536.0, zero))
    shp = (_PB, _N_WIDTH, _N_SAMPLES)
    phi_ref[...] = jnp.broadcast_to(phi_row[:, None, :], shp)
    dphi_ref[...] = jnp.broadcast_to(dphi_row[:, None, :], shp)

    wb = w_ref[...]
    col = g * _PB + lax.broadcasted_iota(_I32, (_PB, _N_WIDTH), 0)
    wb = jnp.where(col < _N_NODES, wb, zero)
    dn = (((0,), (0,)), ((), ()))
    pt = lax.dot_general(wb, phi_row, dn, preferred_element_type=_F32)
    pdt = lax.dot_general(wb, dphi_row, dn, preferred_element_type=_F32)
    pddt = lax.dot_general(wb, ddphi_row, dn, preferred_element_type=_F32)

    @pl.when(g == 0)
    def _init():
        t_ref[...] = pt
        dt_ref[...] = pdt
        ddt_ref[...] = pddt

    @pl.when(g > 0)
    def _acc():
        t_ref[...] = t_ref[...] + pt
        dt_ref[...] = dt_ref[...] + pdt
        ddt_ref[...] = ddt_ref[...] + pddt


@jax.jit
def kernel(x, weight):
    mesh = plsc.VectorSubcoreMesh(core_axis_name="c", subcore_axis_name="s")
    big = jax.ShapeDtypeStruct((_N_NODES, _N_WIDTH, _N_SAMPLES), _F32)
    small = jax.ShapeDtypeStruct((_N_WIDTH, _N_SAMPLES), _F32)
    sc_fn = pl.kernel(
        _sc_body,
        out_type=big,
        mesh=mesh,
        compiler_params=pltpu.CompilerParams(needs_layout_passes=False),
        scratch_types=[
            pltpu.VMEM((_N_SAMPLES,), _F32),
            pltpu.VMEM((8, _N_SAMPLES), _F32),
            pltpu.VMEM((8, _N_SAMPLES), _F32),
            pltpu.VMEM((8, _N_SAMPLES), _F32),
            pltpu.SemaphoreType.DMA,
            pltpu.SemaphoreType.DMA,
            pltpu.SemaphoreType.DMA,
        ],
    )
    ddphi = sc_fn(x)

    nsteps = (_N_NODES + _PB - 1) // _PB
    phi, dphi, t, dt, ddt = pl.pallas_call(
        _tc_body,
        grid=(nsteps,),
        in_specs=[
            pl.BlockSpec((_N_SAMPLES,), lambda g: (0,)),
            pl.BlockSpec((_PB, _N_WIDTH), lambda g: (g, 0)),
        ],
        out_specs=(
            pl.BlockSpec((_PB, _N_WIDTH, _N_SAMPLES), lambda g: (g, 0, 0)),
            pl.BlockSpec((_PB, _N_WIDTH, _N_SAMPLES), lambda g: (g, 0, 0)),
            pl.BlockSpec((_N_WIDTH, _N_SAMPLES), lambda g: (0, 0)),
            pl.BlockSpec((_N_WIDTH, _N_SAMPLES), lambda g: (0, 0)),
            pl.BlockSpec((_N_WIDTH, _N_SAMPLES), lambda g: (0, 0)),
        ),
        out_shape=(big, big, small, small, small),
    )(x, weight.T)

    tr3 = lambda a: jnp.transpose(a, (2, 1, 0))
    return (t.T, dt.T, ddt.T, tr3(phi), tr3(dphi), tr3(ddphi))
